# Initial kernel scaffold; baseline (speedup 1.0000x reference)
import jax
import jax.numpy as jnp
from jax.experimental import pallas as pl


def kernel(z, codebook, codebook_t):
    raise NotImplementedError("write your pallas kernel here")


# speedup vs baseline: 2.6240x; 2.6240x over previous
import functools

import jax
import jax.numpy as jnp
from jax.experimental import pallas as pl
from jax.experimental.pallas import tpu as pltpu

DEPTH = 4
B_TOK = 16384
CODE_DIM = 256
N_CODES = 1024

BLK = 512


def _vq_body(z_ref, cb_ref, ct_ref, zq_ref, m0_ref, m1_ref, m2_ref, m3_ref):
    r = z_ref[...]
    ct = ct_ref[...]
    cb = cb_ref[...]
    ctn = jnp.sum(ct * ct, axis=1)
    maps_refs = (m0_ref, m1_ref, m2_ref, m3_ref)
    zq = jnp.zeros_like(r)
    for i in range(DEPTH):
        rn = jnp.sum(r * r, axis=1, keepdims=True)
        prod = jax.lax.dot_general(
            r, ct, (((1,), (1,)), ((), ())),
            preferred_element_type=jnp.float32)
        dist = rn + ctn[None, :] - 2.0 * prod
        maps_refs[i][...] = dist
        pred = jnp.argmin(dist, axis=1)
        onehot = (jax.lax.broadcasted_iota(jnp.int32, (BLK, N_CODES), 1)
                  == pred[:, None]).astype(jnp.float32)
        delta = jax.lax.dot_general(
            onehot, cb, (((1,), (0,)), ((), ())),
            preferred_element_type=jnp.float32)
        zq = zq + delta
        r = r - delta
    zq_ref[...] = zq


@jax.jit
def kernel(z, codebook, codebook_t):
    grid = (B_TOK // BLK,)
    row_block = pl.BlockSpec((BLK, CODE_DIM), lambda i: (i, 0))
    full_cb = pl.BlockSpec((N_CODES, CODE_DIM), lambda i: (0, 0))
    map_block = pl.BlockSpec((BLK, N_CODES), lambda i: (i, 0))
    out_shapes = (
        jax.ShapeDtypeStruct((B_TOK, CODE_DIM), jnp.float32),
        *(jax.ShapeDtypeStruct((B_TOK, N_CODES), jnp.float32),) * DEPTH,
    )
    zq, m0, m1, m2, m3 = pl.pallas_call(
        _vq_body,
        grid=grid,
        in_specs=[row_block, full_cb, full_cb],
        out_specs=(row_block, *(map_block,) * DEPTH),
        out_shape=out_shapes,
        compiler_params=pltpu.CompilerParams(
            dimension_semantics=("parallel",)),
    )(z, codebook, codebook_t)
    return (zq, m0, m1, m2, m3)


# speedup vs baseline: 3.2288x; 1.2305x over previous
import functools

import jax
import jax.numpy as jnp
from jax.experimental import pallas as pl
from jax.experimental.pallas import tpu as pltpu

DEPTH = 4
B_TOK = 16384
CODE_DIM = 256
N_CODES = 1024

BLK = 1024


def _vq_body(z_ref, cb_ref, ct_ref, zq_ref, m0_ref, m1_ref, m2_ref, m3_ref):
    r = z_ref[...]
    ct = ct_ref[...]
    cb = cb_ref[...]
    ctn = jnp.sum(ct * ct, axis=1)
    maps_refs = (m0_ref, m1_ref, m2_ref, m3_ref)
    zq = jnp.zeros_like(r)
    for i in range(DEPTH):
        rn = jnp.sum(r * r, axis=1, keepdims=True)
        prod = jax.lax.dot_general(
            r, ct, (((1,), (1,)), ((), ())),
            preferred_element_type=jnp.float32)
        dist = rn + ctn[None, :] - 2.0 * prod
        maps_refs[i][...] = dist
        pred = jnp.argmin(dist, axis=1)
        onehot = (jax.lax.broadcasted_iota(jnp.int32, (BLK, N_CODES), 1)
                  == pred[:, None]).astype(jnp.float32)
        delta = jax.lax.dot_general(
            onehot, cb, (((1,), (0,)), ((), ())),
            preferred_element_type=jnp.float32)
        zq = zq + delta
        r = r - delta
    zq_ref[...] = zq


@jax.jit
def kernel(z, codebook, codebook_t):
    grid = (B_TOK // BLK,)
    row_block = pl.BlockSpec((BLK, CODE_DIM), lambda i: (i, 0))
    full_cb = pl.BlockSpec((N_CODES, CODE_DIM), lambda i: (0, 0))
    map_block = pl.BlockSpec((BLK, N_CODES), lambda i: (i, 0))
    out_shapes = (
        jax.ShapeDtypeStruct((B_TOK, CODE_DIM), jnp.float32),
        *(jax.ShapeDtypeStruct((B_TOK, N_CODES), jnp.float32),) * DEPTH,
    )
    zq, m0, m1, m2, m3 = pl.pallas_call(
        _vq_body,
        grid=grid,
        in_specs=[row_block, full_cb, full_cb],
        out_specs=(row_block, *(map_block,) * DEPTH),
        out_shape=out_shapes,
        compiler_params=pltpu.CompilerParams(
            dimension_semantics=("parallel",)),
    )(z, codebook, codebook_t)
    return (zq, m0, m1, m2, m3)
